# Initial kernel scaffold; baseline (speedup 1.0000x reference)
import jax
import jax.numpy as jnp
from jax.experimental import pallas as pl


def kernel(indices, table):
    raise NotImplementedError("write your pallas kernel here")


# speedup vs baseline: 1.1100x; 1.1100x over previous
import functools

import jax
import jax.numpy as jnp
from jax import lax
from jax.experimental import pallas as pl
from jax.experimental.pallas import tpu as pltpu
from jax.experimental.pallas import tpu_sc as plsc


@functools.lru_cache(maxsize=None)
def _make_gather(V, D, B):
    info = plsc.get_sparse_core_info()
    nw = info.num_cores * info.num_subcores
    b_per_w = B // nw
    C = 1600
    n_chunks = b_per_w // C
    mesh = plsc.VectorSubcoreMesh(core_axis_name="c", subcore_axis_name="s")

    @functools.partial(
        pl.kernel,
        mesh=mesh,
        out_type=jax.ShapeDtypeStruct((B, D), jnp.float32),
        scratch_types=[
            pltpu.VMEM((2, C), jnp.int32),
            pltpu.VMEM((2, C, D), jnp.float32),
            pltpu.SemaphoreType.DMA((2,)),
            pltpu.SemaphoreType.DMA((2,)),
            pltpu.SemaphoreType.DMA((2,)),
        ],
        compiler_params=pltpu.CompilerParams(use_tc_tiling_on_sc=False),
    )
    def gather(idx_hbm, table_hbm, out_hbm, idx_v, rows_v, sem_i, sem_g, sem_w):
        wid = lax.axis_index("s") * info.num_cores + lax.axis_index("c")
        base = wid * b_per_w

        def issue_idx(i, b):
            pltpu.async_copy(idx_hbm.at[pl.ds(base + i * C, C)], idx_v.at[b], sem_i.at[b])

        def wait_idx(i, b):
            pltpu.make_async_copy(
                idx_hbm.at[pl.ds(base + i * C, C)], idx_v.at[b], sem_i.at[b]
            ).wait()

        def issue_gather(b):
            pltpu.async_copy(table_hbm.at[idx_v.at[b]], rows_v.at[b], sem_g.at[b])

        def wait_gather(b):
            pltpu.make_async_copy(
                table_hbm.at[idx_v.at[b]], rows_v.at[b], sem_g.at[b]
            ).wait()

        def issue_wb(i, b):
            pltpu.async_copy(rows_v.at[b], out_hbm.at[pl.ds(base + i * C, C)], sem_w.at[b])

        def wait_wb(i, b):
            pltpu.make_async_copy(
                rows_v.at[b], out_hbm.at[pl.ds(base + i * C, C)], sem_w.at[b]
            ).wait()

        issue_idx(0, 0)
        issue_idx(1, 1)
        wait_idx(0, 0)
        issue_gather(0)

        @pl.loop(1, n_chunks)
        def _chunk(i):
            b = lax.rem(i, 2)
            pb = 1 - b
            wait_gather(pb)
            issue_wb(i - 1, pb)
            wait_idx(i, b)

            @pl.when(i + 1 < n_chunks)
            def _():
                issue_idx(i + 1, pb)

            @pl.when(i >= 2)
            def _():
                wait_wb(i - 2, b)

            issue_gather(b)

        last = n_chunks - 1
        lb = last % 2
        wait_gather(lb)
        issue_wb(last, lb)
        wait_wb(last - 1, 1 - lb)
        wait_wb(last, lb)

    return gather


def kernel(indices, table):
    bsz, hist = indices.shape
    V, D = table.shape
    flat = indices.reshape(-1).astype(jnp.int32)
    out = _make_gather(V, D, flat.shape[0])(flat, table)
    return out.reshape(bsz, hist, D)


# speedup vs baseline: 1.1126x; 1.0024x over previous
import functools

import jax
import jax.numpy as jnp
from jax import lax
from jax.experimental import pallas as pl
from jax.experimental.pallas import tpu as pltpu
from jax.experimental.pallas import tpu_sc as plsc

NBUF = 3


@functools.lru_cache(maxsize=None)
def _make_gather(V, D, B):
    info = plsc.get_sparse_core_info()
    nw = info.num_cores * info.num_subcores
    b_per_w = B // nw
    C = 1024
    n_chunks = b_per_w // C
    mesh = plsc.VectorSubcoreMesh(core_axis_name="c", subcore_axis_name="s")

    @functools.partial(
        pl.kernel,
        mesh=mesh,
        out_type=jax.ShapeDtypeStruct((B, D), jnp.float32),
        scratch_types=[
            pltpu.VMEM((NBUF, C), jnp.int32),
            pltpu.VMEM((NBUF, C, D), jnp.float32),
            pltpu.SemaphoreType.DMA((NBUF,)),
            pltpu.SemaphoreType.DMA((NBUF,)),
            pltpu.SemaphoreType.DMA((NBUF,)),
        ],
        compiler_params=pltpu.CompilerParams(use_tc_tiling_on_sc=False),
    )
    def gather(idx_hbm, table_hbm, out_hbm, idx_v, rows_v, sem_i, sem_g, sem_w):
        wid = lax.axis_index("s") * info.num_cores + lax.axis_index("c")
        base = wid * b_per_w

        def issue_idx(i, b):
            pltpu.async_copy(idx_hbm.at[pl.ds(base + i * C, C)], idx_v.at[b], sem_i.at[b])

        def wait_idx(i, b):
            pltpu.make_async_copy(
                idx_hbm.at[pl.ds(base + i * C, C)], idx_v.at[b], sem_i.at[b]
            ).wait()

        def issue_gather(b):
            pltpu.async_copy(table_hbm.at[idx_v.at[b]], rows_v.at[b], sem_g.at[b])

        def wait_gather(b):
            pltpu.make_async_copy(
                table_hbm.at[idx_v.at[b]], rows_v.at[b], sem_g.at[b]
            ).wait()

        def issue_wb(i, b):
            pltpu.async_copy(rows_v.at[b], out_hbm.at[pl.ds(base + i * C, C)], sem_w.at[b])

        def wait_wb(i, b):
            pltpu.make_async_copy(
                rows_v.at[b], out_hbm.at[pl.ds(base + i * C, C)], sem_w.at[b]
            ).wait()

        issue_idx(0, 0)
        issue_idx(1, 1)
        issue_idx(2, 2)
        wait_idx(0, 0)
        issue_gather(0)
        wait_idx(1, 1)
        issue_gather(1)

        @pl.loop(0, n_chunks)
        def _chunk(i):
            b = lax.rem(i, NBUF)
            nb = lax.rem(i + 2, NBUF)
            wait_gather(b)
            issue_wb(i, b)

            @pl.when(i + 2 < n_chunks)
            def _():
                wait_idx(i + 2, nb)

                @pl.when(i >= 1)
                def _():
                    wait_wb(i - 1, nb)

                issue_gather(nb)

            @pl.when(i + 3 < n_chunks)
            def _():
                issue_idx(i + 3, b)

        wait_wb(n_chunks - 3, (n_chunks - 3) % NBUF)
        wait_wb(n_chunks - 2, (n_chunks - 2) % NBUF)
        wait_wb(n_chunks - 1, (n_chunks - 1) % NBUF)

    return gather


def kernel(indices, table):
    bsz, hist = indices.shape
    V, D = table.shape
    flat = indices.reshape(-1).astype(jnp.int32)
    out = _make_gather(V, D, flat.shape[0])(flat, table)
    return out.reshape(bsz, hist, D)


# speedup vs baseline: 1.1142x; 1.0015x over previous
import functools

import jax
import jax.numpy as jnp
from jax import lax
from jax.experimental import pallas as pl
from jax.experimental.pallas import tpu as pltpu
from jax.experimental.pallas import tpu_sc as plsc

NBUF = 3


@functools.lru_cache(maxsize=None)
def _make_gather(V, D, B):
    info = plsc.get_sparse_core_info()
    nw = info.num_cores * info.num_subcores
    b_per_w = B // nw
    C = 1280
    n_chunks = b_per_w // C
    mesh = plsc.VectorSubcoreMesh(core_axis_name="c", subcore_axis_name="s")

    @functools.partial(
        pl.kernel,
        mesh=mesh,
        out_type=jax.ShapeDtypeStruct((B, D), jnp.float32),
        scratch_types=[
            pltpu.VMEM((NBUF, C), jnp.int32),
            pltpu.VMEM((NBUF, C, D), jnp.float32),
            pltpu.SemaphoreType.DMA((NBUF,)),
            pltpu.SemaphoreType.DMA((NBUF,)),
            pltpu.SemaphoreType.DMA((NBUF,)),
        ],
        compiler_params=pltpu.CompilerParams(use_tc_tiling_on_sc=False),
    )
    def gather(idx_hbm, table_hbm, out_hbm, idx_v, rows_v, sem_i, sem_g, sem_w):
        wid = lax.axis_index("s") * info.num_cores + lax.axis_index("c")
        base = wid * b_per_w

        def issue_idx(i, b):
            pltpu.async_copy(idx_hbm.at[pl.ds(base + i * C, C)], idx_v.at[b], sem_i.at[b])

        def wait_idx(i, b):
            pltpu.make_async_copy(
                idx_hbm.at[pl.ds(base + i * C, C)], idx_v.at[b], sem_i.at[b]
            ).wait()

        def issue_gather(b):
            pltpu.async_copy(table_hbm.at[idx_v.at[b]], rows_v.at[b], sem_g.at[b])

        def wait_gather(b):
            pltpu.make_async_copy(
                table_hbm.at[idx_v.at[b]], rows_v.at[b], sem_g.at[b]
            ).wait()

        def issue_wb(i, b):
            pltpu.async_copy(rows_v.at[b], out_hbm.at[pl.ds(base + i * C, C)], sem_w.at[b])

        def wait_wb(i, b):
            pltpu.make_async_copy(
                rows_v.at[b], out_hbm.at[pl.ds(base + i * C, C)], sem_w.at[b]
            ).wait()

        issue_idx(0, 0)
        issue_idx(1, 1)
        issue_idx(2, 2)
        wait_idx(0, 0)
        issue_gather(0)
        wait_idx(1, 1)
        issue_gather(1)

        @pl.loop(0, n_chunks)
        def _chunk(i):
            b = lax.rem(i, NBUF)
            nb = lax.rem(i + 2, NBUF)
            wait_gather(b)
            issue_wb(i, b)

            @pl.when(i + 2 < n_chunks)
            def _():
                wait_idx(i + 2, nb)

                @pl.when(i >= 1)
                def _():
                    wait_wb(i - 1, nb)

                issue_gather(nb)

            @pl.when(i + 3 < n_chunks)
            def _():
                issue_idx(i + 3, b)

        wait_wb(n_chunks - 3, (n_chunks - 3) % NBUF)
        wait_wb(n_chunks - 2, (n_chunks - 2) % NBUF)
        wait_wb(n_chunks - 1, (n_chunks - 1) % NBUF)

    return gather


def kernel(indices, table):
    bsz, hist = indices.shape
    V, D = table.shape
    flat = indices.reshape(-1).astype(jnp.int32)
    out = _make_gather(V, D, flat.shape[0])(flat, table)
    return out.reshape(bsz, hist, D)
